# baseline (speedup 1.0000x reference)
import functools

import jax
import jax.numpy as jnp
from jax import lax
from jax.experimental import pallas as pl
from jax.experimental.pallas import tpu as pltpu
from jax.experimental.pallas import tpu_sc as plsc

_N = 10000
_E = 320000
_D = 128
_L = 4

_W = 128
_HW = _W // 2
_SB = 1
_EB = 2000
_NB = 2000

_f32 = jnp.float32
_bf16 = jnp.bfloat16


def _bdot(a, b):
    return jnp.dot(a.astype(_bf16), b.astype(_bf16),
                   preferred_element_type=_f32)


def _sc_mesh():
    return plsc.VectorSubcoreMesh(core_axis_name="c", subcore_axis_name="s")


def _sc_gather2(ps, pd, src2d, dst2d):
    @functools.partial(
        pl.kernel,
        out_type=(jax.ShapeDtypeStruct((_E, _D), _f32),
                  jax.ShapeDtypeStruct((_E, _D), _f32)),
        mesh=_sc_mesh(),
        scratch_types=[pltpu.SemaphoreType.DMA, pltpu.SemaphoreType.DMA,
                       pltpu.SemaphoreType.DMA, pltpu.SemaphoreType.DMA],
    )
    def k(ps_hbm, pd_hbm, src_hbm, dst_hbm, gs_hbm, gd_hbm,
          sem1, sem2, sem3, sem4):
        del sem3, sem4

        def body(si_v, di_v, gs_v, gd_v):
            c1 = pltpu.async_copy(ps_hbm.at[si_v.at[0]], gs_v, sem1)
            c2 = pltpu.async_copy(pd_hbm.at[di_v.at[0]], gd_v, sem2)
            c1.wait()
            c2.wait()

        pltpu.emit_pipeline(
            body,
            grid=(_E // _W,),
            in_specs=[pl.BlockSpec((1, _W), lambda i: (0, i)),
                      pl.BlockSpec((1, _W), lambda i: (0, i))],
            out_specs=[pl.BlockSpec((_W, _D), lambda i: (i, 0)),
                       pl.BlockSpec((_W, _D), lambda i: (i, 0))],
            core_axis_name=("c", "s"),
            dimension_semantics=(pltpu.PARALLEL,),
        )(src_hbm, dst_hbm, gs_hbm, gd_hbm)

    return k(ps, pd, src2d, dst2d)


def _sc_scatter(e, dstw, init):
    @functools.partial(
        pl.kernel,
        out_type=jax.ShapeDtypeStruct((2, _N, _D), _f32),
        mesh=_sc_mesh(),
        scratch_types=[pltpu.VMEM_SHARED((_N, _D), _f32),
                       pltpu.SemaphoreType.DMA],
    )
    def k(e_hbm, dst_hbm, z_hbm, out_hbm, acc_shared, sem):
        cid = lax.axis_index("c")
        sid = lax.axis_index("s")

        @pl.when(sid == 0)
        def _():
            pltpu.async_copy(z_hbm.at[cid], acc_shared, sem).wait()

        plsc.subcore_barrier()

        def body(e_v, di_v):
            for t in range(_SB):
                pltpu.sync_copy(e_v.at[pl.ds(t * _W, _W)],
                                acc_shared.at[di_v.at[t]], add=True)

        pltpu.emit_pipeline(
            body,
            grid=(_E // (_SB * _W),),
            in_specs=[pl.BlockSpec((_SB * _W, _D), lambda i: (i, 0)),
                      pl.BlockSpec((_SB, _W), lambda i: (i, 0))],
            out_specs=[],
            core_axis_name=("c", "s"),
            dimension_semantics=(pltpu.PARALLEL,),
        )(e_hbm, dst_hbm)

        plsc.subcore_barrier()

        rows = 1000

        @pl.when(sid < 10)
        def _():
            pltpu.async_copy(acc_shared.at[pl.ds(sid * rows, rows)],
                             out_hbm.at[cid, pl.ds(sid * rows, rows)],
                             sem).wait()

    return k(e, dstw, init)


def _ln_res(x, h, ls, lb):
    mu = jnp.mean(h, axis=-1, keepdims=True)
    hc = h - mu
    var = jnp.mean(hc * hc, axis=-1, keepdims=True)
    return x + ls * hc * lax.rsqrt(var + 1e-5) + lb


def _tc_edge(e, gs, gd, w1e, b1, w2, b2, ls, lb):
    def body(e_ref, gs_ref, gd_ref, w1e_ref, b1_ref, w2_ref, b2_ref,
             ls_ref, lb_ref, o_ref):
        x = e_ref[...]
        h = _bdot(x, w1e_ref[...])
        h = h + gs_ref[...] + gd_ref[...] + b1_ref[...]
        h = h * lax.logistic(h)
        h = _bdot(h, w2_ref[...]) + b2_ref[...]
        o_ref[...] = _ln_res(x, h, ls_ref[...], lb_ref[...])

    row = pl.BlockSpec((_EB, _D), lambda i: (i, 0))
    full = pl.BlockSpec((_D, _D), lambda i: (0, 0))
    vec = pl.BlockSpec((1, _D), lambda i: (0, 0))
    return pl.pallas_call(
        body,
        grid=(_E // _EB,),
        in_specs=[row, row, row, full, vec, full, vec, vec, vec],
        out_specs=row,
        out_shape=jax.ShapeDtypeStruct((_E, _D), _f32),
    )(e, gs, gd, w1e, b1, w2, b2, ls, lb)


def _tc_node(n, parts, w1n, w1a, b1, w2, b2, ls, lb, wps, wpd):
    def body(n_ref, p_ref, w1n_ref, w1a_ref, b1_ref, w2_ref, b2_ref,
             ls_ref, lb_ref, wps_ref, wpd_ref, o_ref, ps_ref, pd_ref):
        x = n_ref[...]
        agg = p_ref[0] + p_ref[1]
        h = _bdot(x, w1n_ref[...]) + _bdot(agg, w1a_ref[...]) + b1_ref[...]
        h = h * lax.logistic(h)
        h = _bdot(h, w2_ref[...]) + b2_ref[...]
        nn = _ln_res(x, h, ls_ref[...], lb_ref[...])
        o_ref[...] = nn
        ps_ref[...] = _bdot(nn, wps_ref[...])
        pd_ref[...] = _bdot(nn, wpd_ref[...])

    row = pl.BlockSpec((_NB, _D), lambda i: (i, 0))
    prow = pl.BlockSpec((2, _NB, _D), lambda i: (0, i, 0))
    full = pl.BlockSpec((_D, _D), lambda i: (0, 0))
    vec = pl.BlockSpec((1, _D), lambda i: (0, 0))
    shp = jax.ShapeDtypeStruct((_N, _D), _f32)
    return pl.pallas_call(
        body,
        grid=(_N // _NB,),
        in_specs=[row, prow, full, full, vec, full, vec, vec, vec, full, full],
        out_specs=[row, row, row],
        out_shape=[shp, shp, shp],
    )(n, parts, w1n, w1a, b1, w2, b2, ls, lb, wps, wpd)


def _tc_proj(n, wps, wpd):
    def body(n_ref, wps_ref, wpd_ref, ps_ref, pd_ref):
        x = n_ref[...]
        ps_ref[...] = _bdot(x, wps_ref[...])
        pd_ref[...] = _bdot(x, wpd_ref[...])

    row = pl.BlockSpec((_NB, _D), lambda i: (i, 0))
    full = pl.BlockSpec((_D, _D), lambda i: (0, 0))
    shp = jax.ShapeDtypeStruct((_N, _D), _f32)
    return pl.pallas_call(
        body,
        grid=(_N // _NB,),
        in_specs=[row, full, full],
        out_specs=[row, row],
        out_shape=[shp, shp],
    )(n, wps, wpd)


def kernel(embedded_mesh_features, embedded_mesh2mesh_edge_features,
           mesh2mesh_edge_indices_src, mesh2mesh_edge_indices_dst,
           edge_w1, edge_b1, edge_w2, edge_b2, edge_ln_scale, edge_ln_bias,
           node_w1, node_b1, node_w2, node_b2, node_ln_scale, node_ln_bias):
    n = embedded_mesh_features
    e = embedded_mesh2mesh_edge_features
    src2d = mesh2mesh_edge_indices_src.reshape(1, _E)
    dst2d = mesh2mesh_edge_indices_dst.reshape(1, _E)
    dstw = mesh2mesh_edge_indices_dst.reshape(_E // _W, _W)
    zeros2 = jnp.zeros((2, _N, _D), _f32)

    w1e = [edge_w1[l, :_D] for l in range(_L)]
    w1s = [edge_w1[l, _D:2 * _D] for l in range(_L)]
    w1d = [edge_w1[l, 2 * _D:] for l in range(_L)]
    eb1 = [edge_b1[l].reshape(1, _D) for l in range(_L)]
    eb2 = [edge_b2[l].reshape(1, _D) for l in range(_L)]
    els = [edge_ln_scale[l].reshape(1, _D) for l in range(_L)]
    elb = [edge_ln_bias[l].reshape(1, _D) for l in range(_L)]
    w1n = [node_w1[l, :_D] for l in range(_L)]
    w1a = [node_w1[l, _D:] for l in range(_L)]
    nb1 = [node_b1[l].reshape(1, _D) for l in range(_L)]
    nb2 = [node_b2[l].reshape(1, _D) for l in range(_L)]
    nls = [node_ln_scale[l].reshape(1, _D) for l in range(_L)]
    nlb = [node_ln_bias[l].reshape(1, _D) for l in range(_L)]

    ps, pd = _tc_proj(n, w1s[0], w1d[0])
    for l in range(_L):
        gs, gd = _sc_gather2(ps, pd, src2d, dst2d)
        e = _tc_edge(e, gs, gd, w1e[l], eb1[l], edge_w2[l], eb2[l],
                     els[l], elb[l])
        parts = _sc_scatter(e, dstw, zeros2)
        nxt = (l + 1) % _L
        n, ps, pd = _tc_node(n, parts, w1n[l], w1a[l], nb1[l], node_w2[l],
                             nb2[l], nls[l], nlb[l], w1s[nxt], w1d[nxt])
    return (n, e)


# speedup vs baseline: 1.0168x; 1.0168x over previous
import functools

import jax
import jax.numpy as jnp
from jax import lax
from jax.experimental import pallas as pl
from jax.experimental.pallas import tpu as pltpu
from jax.experimental.pallas import tpu_sc as plsc

_N = 10000
_E = 320000
_D = 128
_L = 4

_W = 128
_HW = _W // 2
_SB = 1
_EB = 2000
_NB = 2000

_f32 = jnp.float32
_bf16 = jnp.bfloat16


def _bdot(a, b):
    return jnp.dot(a, b, preferred_element_type=_f32)


def _sc_mesh():
    return plsc.VectorSubcoreMesh(core_axis_name="c", subcore_axis_name="s")


def _sc_gather2(ps, pd, src2d, dst2d):
    @functools.partial(
        pl.kernel,
        out_type=(jax.ShapeDtypeStruct((_E, _D), _f32),
                  jax.ShapeDtypeStruct((_E, _D), _f32)),
        mesh=_sc_mesh(),
        scratch_types=[pltpu.SemaphoreType.DMA, pltpu.SemaphoreType.DMA,
                       pltpu.SemaphoreType.DMA, pltpu.SemaphoreType.DMA],
    )
    def k(ps_hbm, pd_hbm, src_hbm, dst_hbm, gs_hbm, gd_hbm,
          sem1, sem2, sem3, sem4):
        del sem3, sem4

        def body(si_v, di_v, gs_v, gd_v):
            c1 = pltpu.async_copy(ps_hbm.at[si_v.at[0]], gs_v, sem1)
            c2 = pltpu.async_copy(pd_hbm.at[di_v.at[0]], gd_v, sem2)
            c1.wait()
            c2.wait()

        pltpu.emit_pipeline(
            body,
            grid=(_E // _W,),
            in_specs=[pl.BlockSpec((1, _W), lambda i: (0, i)),
                      pl.BlockSpec((1, _W), lambda i: (0, i))],
            out_specs=[pl.BlockSpec((_W, _D), lambda i: (i, 0)),
                       pl.BlockSpec((_W, _D), lambda i: (i, 0))],
            core_axis_name=("c", "s"),
            dimension_semantics=(pltpu.PARALLEL,),
        )(src_hbm, dst_hbm, gs_hbm, gd_hbm)

    return k(ps, pd, src2d, dst2d)


def _sc_scatter(e, dstw, init):
    @functools.partial(
        pl.kernel,
        out_type=jax.ShapeDtypeStruct((2, _N, _D), _f32),
        mesh=_sc_mesh(),
        scratch_types=[pltpu.VMEM_SHARED((_N, _D), _f32),
                       pltpu.SemaphoreType.DMA],
    )
    def k(e_hbm, dst_hbm, z_hbm, out_hbm, acc_shared, sem):
        cid = lax.axis_index("c")
        sid = lax.axis_index("s")

        @pl.when(sid == 0)
        def _():
            pltpu.async_copy(z_hbm.at[cid], acc_shared, sem).wait()

        plsc.subcore_barrier()

        def body(e_v, di_v):
            for t in range(_SB):
                pltpu.sync_copy(e_v.at[pl.ds(t * _W, _W)],
                                acc_shared.at[di_v.at[t]], add=True)

        pltpu.emit_pipeline(
            body,
            grid=(_E // (_SB * _W),),
            in_specs=[pl.BlockSpec((_SB * _W, _D), lambda i: (i, 0)),
                      pl.BlockSpec((_SB, _W), lambda i: (i, 0))],
            out_specs=[],
            core_axis_name=("c", "s"),
            dimension_semantics=(pltpu.PARALLEL,),
        )(e_hbm, dst_hbm)

        plsc.subcore_barrier()

        rows = 1000

        @pl.when(sid < 10)
        def _():
            pltpu.async_copy(acc_shared.at[pl.ds(sid * rows, rows)],
                             out_hbm.at[cid, pl.ds(sid * rows, rows)],
                             sem).wait()

    return k(e, dstw, init)


def _ln_res(x, h, ls, lb):
    mu = jnp.mean(h, axis=-1, keepdims=True)
    hc = h - mu
    var = jnp.mean(hc * hc, axis=-1, keepdims=True)
    return x + ls * hc * lax.rsqrt(var + 1e-5) + lb


def _tc_edge(e, gs, gd, w1e, b1, w2, b2, ls, lb):
    def body(e_ref, gs_ref, gd_ref, w1e_ref, b1_ref, w2_ref, b2_ref,
             ls_ref, lb_ref, o_ref):
        x = e_ref[...]
        h = _bdot(x, w1e_ref[...])
        h = h + gs_ref[...] + gd_ref[...] + b1_ref[...]
        h = h * lax.logistic(h)
        h = _bdot(h, w2_ref[...]) + b2_ref[...]
        o_ref[...] = _ln_res(x, h, ls_ref[...], lb_ref[...])

    row = pl.BlockSpec((_EB, _D), lambda i: (i, 0))
    full = pl.BlockSpec((_D, _D), lambda i: (0, 0))
    vec = pl.BlockSpec((1, _D), lambda i: (0, 0))
    return pl.pallas_call(
        body,
        grid=(_E // _EB,),
        in_specs=[row, row, row, full, vec, full, vec, vec, vec],
        out_specs=row,
        out_shape=jax.ShapeDtypeStruct((_E, _D), _f32),
    )(e, gs, gd, w1e, b1, w2, b2, ls, lb)


def _tc_node(n, parts, w1n, w1a, b1, w2, b2, ls, lb, wps, wpd):
    def body(n_ref, p_ref, w1n_ref, w1a_ref, b1_ref, w2_ref, b2_ref,
             ls_ref, lb_ref, wps_ref, wpd_ref, o_ref, ps_ref, pd_ref):
        x = n_ref[...]
        agg = p_ref[0] + p_ref[1]
        h = _bdot(x, w1n_ref[...]) + _bdot(agg, w1a_ref[...]) + b1_ref[...]
        h = h * lax.logistic(h)
        h = _bdot(h, w2_ref[...]) + b2_ref[...]
        nn = _ln_res(x, h, ls_ref[...], lb_ref[...])
        o_ref[...] = nn
        ps_ref[...] = _bdot(nn, wps_ref[...])
        pd_ref[...] = _bdot(nn, wpd_ref[...])

    row = pl.BlockSpec((_NB, _D), lambda i: (i, 0))
    prow = pl.BlockSpec((2, _NB, _D), lambda i: (0, i, 0))
    full = pl.BlockSpec((_D, _D), lambda i: (0, 0))
    vec = pl.BlockSpec((1, _D), lambda i: (0, 0))
    shp = jax.ShapeDtypeStruct((_N, _D), _f32)
    return pl.pallas_call(
        body,
        grid=(_N // _NB,),
        in_specs=[row, prow, full, full, vec, full, vec, vec, vec, full, full],
        out_specs=[row, row, row],
        out_shape=[shp, shp, shp],
    )(n, parts, w1n, w1a, b1, w2, b2, ls, lb, wps, wpd)


def _tc_proj(n, wps, wpd):
    def body(n_ref, wps_ref, wpd_ref, ps_ref, pd_ref):
        x = n_ref[...]
        ps_ref[...] = _bdot(x, wps_ref[...])
        pd_ref[...] = _bdot(x, wpd_ref[...])

    row = pl.BlockSpec((_NB, _D), lambda i: (i, 0))
    full = pl.BlockSpec((_D, _D), lambda i: (0, 0))
    shp = jax.ShapeDtypeStruct((_N, _D), _f32)
    return pl.pallas_call(
        body,
        grid=(_N // _NB,),
        in_specs=[row, full, full],
        out_specs=[row, row],
        out_shape=[shp, shp],
    )(n, wps, wpd)


def kernel(embedded_mesh_features, embedded_mesh2mesh_edge_features,
           mesh2mesh_edge_indices_src, mesh2mesh_edge_indices_dst,
           edge_w1, edge_b1, edge_w2, edge_b2, edge_ln_scale, edge_ln_bias,
           node_w1, node_b1, node_w2, node_b2, node_ln_scale, node_ln_bias):
    n = embedded_mesh_features
    e = embedded_mesh2mesh_edge_features
    src2d = mesh2mesh_edge_indices_src.reshape(1, _E)
    dst2d = mesh2mesh_edge_indices_dst.reshape(1, _E)
    dstw = mesh2mesh_edge_indices_dst.reshape(_E // _W, _W)
    zeros2 = jnp.zeros((2, _N, _D), _f32)

    w1e = [edge_w1[l, :_D] for l in range(_L)]
    w1s = [edge_w1[l, _D:2 * _D] for l in range(_L)]
    w1d = [edge_w1[l, 2 * _D:] for l in range(_L)]
    eb1 = [edge_b1[l].reshape(1, _D) for l in range(_L)]
    eb2 = [edge_b2[l].reshape(1, _D) for l in range(_L)]
    els = [edge_ln_scale[l].reshape(1, _D) for l in range(_L)]
    elb = [edge_ln_bias[l].reshape(1, _D) for l in range(_L)]
    w1n = [node_w1[l, :_D] for l in range(_L)]
    w1a = [node_w1[l, _D:] for l in range(_L)]
    nb1 = [node_b1[l].reshape(1, _D) for l in range(_L)]
    nb2 = [node_b2[l].reshape(1, _D) for l in range(_L)]
    nls = [node_ln_scale[l].reshape(1, _D) for l in range(_L)]
    nlb = [node_ln_bias[l].reshape(1, _D) for l in range(_L)]

    ps, pd = _tc_proj(n, w1s[0], w1d[0])
    for l in range(_L):
        gs, gd = _sc_gather2(ps, pd, src2d, dst2d)
        e = _tc_edge(e, gs, gd, w1e[l], eb1[l], edge_w2[l], eb2[l],
                     els[l], elb[l])
        parts = _sc_scatter(e, dstw, zeros2)
        nxt = (l + 1) % _L
        n, ps, pd = _tc_node(n, parts, w1n[l], w1a[l], nb1[l], node_w2[l],
                             nb2[l], nls[l], nlb[l], w1s[nxt], w1d[nxt])
    return (n, e)


# speedup vs baseline: 1.0181x; 1.0012x over previous
import functools

import jax
import jax.numpy as jnp
from jax import lax
from jax.experimental import pallas as pl
from jax.experimental.pallas import tpu as pltpu
from jax.experimental.pallas import tpu_sc as plsc

_N = 10000
_E = 320000
_D = 128
_L = 4

_W = 128
_HW = _W // 2
_SB = 1
_EB = 2000
_NB = 2000

_f32 = jnp.float32
_bf16 = jnp.bfloat16


def _bdot(a, b):
    return jnp.dot(a, b, preferred_element_type=_f32)


def _sc_mesh():
    return plsc.VectorSubcoreMesh(core_axis_name="c", subcore_axis_name="s")


def _sc_gather2(ps, pd, src2d, dst2d):
    @functools.partial(
        pl.kernel,
        out_type=(jax.ShapeDtypeStruct((_E, _D), _f32),
                  jax.ShapeDtypeStruct((_E, _D), _f32)),
        mesh=_sc_mesh(),
        scratch_types=[pltpu.SemaphoreType.DMA, pltpu.SemaphoreType.DMA],
    )
    def k(ps_hbm, pd_hbm, src_hbm, dst_hbm, gs_hbm, gd_hbm, sem1, sem2):
        def body(si_v, di_v, gs_v, gd_v):
            c1 = pltpu.async_copy(ps_hbm.at[si_v.at[0]], gs_v, sem1)
            c2 = pltpu.async_copy(pd_hbm.at[di_v.at[0]], gd_v, sem2)
            c1.wait()
            c2.wait()

        pltpu.emit_pipeline(
            body,
            grid=(_E // _W,),
            in_specs=[pl.BlockSpec((1, _W), lambda i: (0, i)),
                      pl.BlockSpec((1, _W), lambda i: (0, i))],
            out_specs=[pl.BlockSpec((_W, _D), lambda i: (i, 0)),
                       pl.BlockSpec((_W, _D), lambda i: (i, 0))],
            core_axis_name=("c", "s"),
            dimension_semantics=(pltpu.PARALLEL,),
        )(src_hbm, dst_hbm, gs_hbm, gd_hbm)

    return k(ps, pd, src2d, dst2d)


def _sc_scatter(e, dstw, init):
    @functools.partial(
        pl.kernel,
        out_type=jax.ShapeDtypeStruct((2, _N, _D), _f32),
        mesh=_sc_mesh(),
        scratch_types=[pltpu.VMEM_SHARED((_N, _D), _f32),
                       pltpu.SemaphoreType.DMA],
    )
    def k(e_hbm, dst_hbm, z_hbm, out_hbm, acc_shared, sem):
        cid = lax.axis_index("c")
        sid = lax.axis_index("s")

        @pl.when(sid == 0)
        def _():
            pltpu.async_copy(z_hbm, acc_shared, sem).wait()

        plsc.subcore_barrier()

        def body(e_v, di_v):
            pltpu.sync_copy(e_v, acc_shared.at[di_v.at[0]], add=True)

        pltpu.emit_pipeline(
            body,
            grid=(_E // _W,),
            in_specs=[pl.BlockSpec((_W, _D), lambda i: (i, 0)),
                      pl.BlockSpec((1, _W), lambda i: (0, i))],
            out_specs=[],
            core_axis_name=("c", "s"),
            dimension_semantics=(pltpu.PARALLEL,),
        )(e_hbm, dst_hbm)

        plsc.subcore_barrier()

        rows = 1000

        @pl.when(sid < 10)
        def _():
            pltpu.async_copy(acc_shared.at[pl.ds(sid * rows, rows)],
                             out_hbm.at[cid, pl.ds(sid * rows, rows)],
                             sem).wait()

    return k(e, dstw, init)


def _ln_res(x, h, ls, lb):
    mu = jnp.mean(h, axis=-1, keepdims=True)
    hc = h - mu
    var = jnp.mean(hc * hc, axis=-1, keepdims=True)
    return x + ls * hc * lax.rsqrt(var + 1e-5) + lb


def _tc_edge(e, gs, gd, w1e, b1, w2, b2, ls, lb):
    def body(e_ref, gs_ref, gd_ref, w1e_ref, b1_ref, w2_ref, b2_ref,
             ls_ref, lb_ref, o_ref):
        x = e_ref[...]
        h = _bdot(x, w1e_ref[...])
        h = h + gs_ref[...] + gd_ref[...] + b1_ref[...]
        h = h * lax.logistic(h)
        h = _bdot(h, w2_ref[...]) + b2_ref[...]
        o_ref[...] = _ln_res(x, h, ls_ref[...], lb_ref[...])

    row = pl.BlockSpec((_EB, _D), lambda i: (i, 0))
    full = pl.BlockSpec((_D, _D), lambda i: (0, 0))
    vec = pl.BlockSpec((1, _D), lambda i: (0, 0))
    return pl.pallas_call(
        body,
        grid=(_E // _EB,),
        in_specs=[row, row, row, full, vec, full, vec, vec, vec],
        out_specs=row,
        out_shape=jax.ShapeDtypeStruct((_E, _D), _f32),
    )(e, gs, gd, w1e, b1, w2, b2, ls, lb)


def _tc_node(n, parts, w1n, w1a, b1, w2, b2, ls, lb, wps, wpd):
    def body(n_ref, p_ref, w1n_ref, w1a_ref, b1_ref, w2_ref, b2_ref,
             ls_ref, lb_ref, wps_ref, wpd_ref, o_ref, ps_ref, pd_ref):
        x = n_ref[...]
        agg = p_ref[0] + p_ref[1]
        h = _bdot(x, w1n_ref[...]) + _bdot(agg, w1a_ref[...]) + b1_ref[...]
        h = h * lax.logistic(h)
        h = _bdot(h, w2_ref[...]) + b2_ref[...]
        nn = _ln_res(x, h, ls_ref[...], lb_ref[...])
        o_ref[...] = nn
        ps_ref[...] = _bdot(nn, wps_ref[...])
        pd_ref[...] = _bdot(nn, wpd_ref[...])

    row = pl.BlockSpec((_NB, _D), lambda i: (i, 0))
    prow = pl.BlockSpec((2, _NB, _D), lambda i: (0, i, 0))
    full = pl.BlockSpec((_D, _D), lambda i: (0, 0))
    vec = pl.BlockSpec((1, _D), lambda i: (0, 0))
    shp = jax.ShapeDtypeStruct((_N, _D), _f32)
    return pl.pallas_call(
        body,
        grid=(_N // _NB,),
        in_specs=[row, prow, full, full, vec, full, vec, vec, vec, full, full],
        out_specs=[row, row, row],
        out_shape=[shp, shp, shp],
    )(n, parts, w1n, w1a, b1, w2, b2, ls, lb, wps, wpd)


def _tc_proj(n, wps, wpd):
    def body(n_ref, wps_ref, wpd_ref, ps_ref, pd_ref):
        x = n_ref[...]
        ps_ref[...] = _bdot(x, wps_ref[...])
        pd_ref[...] = _bdot(x, wpd_ref[...])

    row = pl.BlockSpec((_NB, _D), lambda i: (i, 0))
    full = pl.BlockSpec((_D, _D), lambda i: (0, 0))
    shp = jax.ShapeDtypeStruct((_N, _D), _f32)
    return pl.pallas_call(
        body,
        grid=(_N // _NB,),
        in_specs=[row, full, full],
        out_specs=[row, row],
        out_shape=[shp, shp],
    )(n, wps, wpd)


def kernel(embedded_mesh_features, embedded_mesh2mesh_edge_features,
           mesh2mesh_edge_indices_src, mesh2mesh_edge_indices_dst,
           edge_w1, edge_b1, edge_w2, edge_b2, edge_ln_scale, edge_ln_bias,
           node_w1, node_b1, node_w2, node_b2, node_ln_scale, node_ln_bias):
    n = embedded_mesh_features
    e = embedded_mesh2mesh_edge_features
    src2d = mesh2mesh_edge_indices_src.reshape(1, _E)
    dst2d = mesh2mesh_edge_indices_dst.reshape(1, _E)
    zeros_nd = jnp.zeros((_N, _D), _f32)

    w1e = [edge_w1[l, :_D] for l in range(_L)]
    w1s = [edge_w1[l, _D:2 * _D] for l in range(_L)]
    w1d = [edge_w1[l, 2 * _D:] for l in range(_L)]
    eb1 = [edge_b1[l].reshape(1, _D) for l in range(_L)]
    eb2 = [edge_b2[l].reshape(1, _D) for l in range(_L)]
    els = [edge_ln_scale[l].reshape(1, _D) for l in range(_L)]
    elb = [edge_ln_bias[l].reshape(1, _D) for l in range(_L)]
    w1n = [node_w1[l, :_D] for l in range(_L)]
    w1a = [node_w1[l, _D:] for l in range(_L)]
    nb1 = [node_b1[l].reshape(1, _D) for l in range(_L)]
    nb2 = [node_b2[l].reshape(1, _D) for l in range(_L)]
    nls = [node_ln_scale[l].reshape(1, _D) for l in range(_L)]
    nlb = [node_ln_bias[l].reshape(1, _D) for l in range(_L)]

    ps, pd = _tc_proj(n, w1s[0], w1d[0])
    for l in range(_L):
        gs, gd = _sc_gather2(ps, pd, src2d, dst2d)
        e = _tc_edge(e, gs, gd, w1e[l], eb1[l], edge_w2[l], eb2[l],
                     els[l], elb[l])
        parts = _sc_scatter(e, dst2d, zeros_nd)
        nxt = (l + 1) % _L
        n, ps, pd = _tc_node(n, parts, w1n[l], w1a[l], nb1[l], node_w2[l],
                             nb2[l], nls[l], nlb[l], w1s[nxt], w1d[nxt])
    return (n, e)


# speedup vs baseline: 1.0616x; 1.0428x over previous
import functools

import jax
import jax.numpy as jnp
from jax import lax
from jax.experimental import pallas as pl
from jax.experimental.pallas import tpu as pltpu
from jax.experimental.pallas import tpu_sc as plsc

_N = 10000
_E = 320000
_D = 128
_L = 4

_W = 128
_HW = _W // 2
_SB = 1
_EB = 2560
_NB = 2000

_f32 = jnp.float32
_bf16 = jnp.bfloat16


def _bdot(a, b):
    return jnp.dot(a, b, preferred_element_type=_f32)


def _sc_mesh():
    return plsc.VectorSubcoreMesh(core_axis_name="c", subcore_axis_name="s")


def _sc_gather2(ps, pd, src2d, dst2d):
    @functools.partial(
        pl.kernel,
        out_type=(jax.ShapeDtypeStruct((_E, _D), _f32),
                  jax.ShapeDtypeStruct((_E, _D), _f32)),
        mesh=_sc_mesh(),
        scratch_types=[pltpu.SemaphoreType.DMA, pltpu.SemaphoreType.DMA],
    )
    def k(ps_hbm, pd_hbm, src_hbm, dst_hbm, gs_hbm, gd_hbm, sem1, sem2):
        def body(si_v, di_v, gs_v, gd_v):
            c1 = pltpu.async_copy(ps_hbm.at[si_v.at[0]], gs_v, sem1)
            c2 = pltpu.async_copy(pd_hbm.at[di_v.at[0]], gd_v, sem2)
            c1.wait()
            c2.wait()

        pltpu.emit_pipeline(
            body,
            grid=(_E // _W,),
            in_specs=[pl.BlockSpec((1, _W), lambda i: (0, i)),
                      pl.BlockSpec((1, _W), lambda i: (0, i))],
            out_specs=[pl.BlockSpec((_W, _D), lambda i: (i, 0)),
                       pl.BlockSpec((_W, _D), lambda i: (i, 0))],
            core_axis_name=("c", "s"),
            dimension_semantics=(pltpu.PARALLEL,),
        )(src_hbm, dst_hbm, gs_hbm, gd_hbm)

    return k(ps, pd, src2d, dst2d)


def _sc_scatter(e, dstw, init):
    @functools.partial(
        pl.kernel,
        out_type=jax.ShapeDtypeStruct((2, _N, _D), _f32),
        mesh=_sc_mesh(),
        scratch_types=[pltpu.VMEM_SHARED((_N, _D), _f32),
                       pltpu.SemaphoreType.DMA],
    )
    def k(e_hbm, dst_hbm, z_hbm, out_hbm, acc_shared, sem):
        cid = lax.axis_index("c")
        sid = lax.axis_index("s")

        @pl.when(sid == 0)
        def _():
            pltpu.async_copy(z_hbm, acc_shared, sem).wait()

        plsc.subcore_barrier()

        def body(e_v, di_v):
            pltpu.sync_copy(e_v, acc_shared.at[di_v.at[0]], add=True)

        pltpu.emit_pipeline(
            body,
            grid=(_E // _W,),
            in_specs=[pl.BlockSpec((_W, _D), lambda i: (i, 0)),
                      pl.BlockSpec((1, _W), lambda i: (0, i))],
            out_specs=[],
            core_axis_name=("c", "s"),
            dimension_semantics=(pltpu.PARALLEL,),
        )(e_hbm, dst_hbm)

        plsc.subcore_barrier()

        rows = 1000

        @pl.when(sid < 10)
        def _():
            pltpu.async_copy(acc_shared.at[pl.ds(sid * rows, rows)],
                             out_hbm.at[cid, pl.ds(sid * rows, rows)],
                             sem).wait()

    return k(e, dstw, init)


def _ln_res(x, h, ls, lb):
    mu = jnp.mean(h, axis=-1, keepdims=True)
    hc = h - mu
    var = jnp.mean(hc * hc, axis=-1, keepdims=True)
    return x + ls * hc * lax.rsqrt(var + 1e-5) + lb


def _tc_edge(e, gs, gd, w1e, b1, w2, b2, ls, lb):
    def body(e_ref, gs_ref, gd_ref, w1e_ref, b1_ref, w2_ref, b2_ref,
             ls_ref, lb_ref, o_ref):
        x = e_ref[...]
        h = _bdot(x, w1e_ref[...])
        h = h + gs_ref[...] + gd_ref[...] + b1_ref[...]
        h = h * lax.logistic(h)
        h = _bdot(h, w2_ref[...]) + b2_ref[...]
        o_ref[...] = _ln_res(x, h, ls_ref[...], lb_ref[...])

    row = pl.BlockSpec((_EB, _D), lambda i: (i, 0))
    full = pl.BlockSpec((_D, _D), lambda i: (0, 0))
    vec = pl.BlockSpec((1, _D), lambda i: (0, 0))
    return pl.pallas_call(
        body,
        grid=(_E // _EB,),
        in_specs=[row, row, row, full, vec, full, vec, vec, vec],
        out_specs=row,
        out_shape=jax.ShapeDtypeStruct((_E, _D), _f32),
    )(e, gs, gd, w1e, b1, w2, b2, ls, lb)


def _tc_node(n, parts, w1n, w1a, b1, w2, b2, ls, lb, wps, wpd):
    def body(n_ref, p_ref, w1n_ref, w1a_ref, b1_ref, w2_ref, b2_ref,
             ls_ref, lb_ref, wps_ref, wpd_ref, o_ref, ps_ref, pd_ref):
        x = n_ref[...]
        agg = p_ref[0] + p_ref[1]
        h = _bdot(x, w1n_ref[...]) + _bdot(agg, w1a_ref[...]) + b1_ref[...]
        h = h * lax.logistic(h)
        h = _bdot(h, w2_ref[...]) + b2_ref[...]
        nn = _ln_res(x, h, ls_ref[...], lb_ref[...])
        o_ref[...] = nn
        ps_ref[...] = _bdot(nn, wps_ref[...])
        pd_ref[...] = _bdot(nn, wpd_ref[...])

    row = pl.BlockSpec((_NB, _D), lambda i: (i, 0))
    prow = pl.BlockSpec((2, _NB, _D), lambda i: (0, i, 0))
    full = pl.BlockSpec((_D, _D), lambda i: (0, 0))
    vec = pl.BlockSpec((1, _D), lambda i: (0, 0))
    shp = jax.ShapeDtypeStruct((_N, _D), _f32)
    return pl.pallas_call(
        body,
        grid=(_N // _NB,),
        in_specs=[row, prow, full, full, vec, full, vec, vec, vec, full, full],
        out_specs=[row, row, row],
        out_shape=[shp, shp, shp],
    )(n, parts, w1n, w1a, b1, w2, b2, ls, lb, wps, wpd)


def _tc_proj(n, wps, wpd):
    def body(n_ref, wps_ref, wpd_ref, ps_ref, pd_ref):
        x = n_ref[...]
        ps_ref[...] = _bdot(x, wps_ref[...])
        pd_ref[...] = _bdot(x, wpd_ref[...])

    row = pl.BlockSpec((_NB, _D), lambda i: (i, 0))
    full = pl.BlockSpec((_D, _D), lambda i: (0, 0))
    shp = jax.ShapeDtypeStruct((_N, _D), _f32)
    return pl.pallas_call(
        body,
        grid=(_N // _NB,),
        in_specs=[row, full, full],
        out_specs=[row, row],
        out_shape=[shp, shp],
    )(n, wps, wpd)


def kernel(embedded_mesh_features, embedded_mesh2mesh_edge_features,
           mesh2mesh_edge_indices_src, mesh2mesh_edge_indices_dst,
           edge_w1, edge_b1, edge_w2, edge_b2, edge_ln_scale, edge_ln_bias,
           node_w1, node_b1, node_w2, node_b2, node_ln_scale, node_ln_bias):
    n = embedded_mesh_features
    e = embedded_mesh2mesh_edge_features
    src2d = mesh2mesh_edge_indices_src.reshape(1, _E)
    dst2d = mesh2mesh_edge_indices_dst.reshape(1, _E)
    zeros_nd = jnp.zeros((_N, _D), _f32)

    w1e = [edge_w1[l, :_D] for l in range(_L)]
    w1s = [edge_w1[l, _D:2 * _D] for l in range(_L)]
    w1d = [edge_w1[l, 2 * _D:] for l in range(_L)]
    eb1 = [edge_b1[l].reshape(1, _D) for l in range(_L)]
    eb2 = [edge_b2[l].reshape(1, _D) for l in range(_L)]
    els = [edge_ln_scale[l].reshape(1, _D) for l in range(_L)]
    elb = [edge_ln_bias[l].reshape(1, _D) for l in range(_L)]
    w1n = [node_w1[l, :_D] for l in range(_L)]
    w1a = [node_w1[l, _D:] for l in range(_L)]
    nb1 = [node_b1[l].reshape(1, _D) for l in range(_L)]
    nb2 = [node_b2[l].reshape(1, _D) for l in range(_L)]
    nls = [node_ln_scale[l].reshape(1, _D) for l in range(_L)]
    nlb = [node_ln_bias[l].reshape(1, _D) for l in range(_L)]

    ps, pd = _tc_proj(n, w1s[0], w1d[0])
    for l in range(_L):
        gs, gd = _sc_gather2(ps, pd, src2d, dst2d)
        e = _tc_edge(e, gs, gd, w1e[l], eb1[l], edge_w2[l], eb2[l],
                     els[l], elb[l])
        parts = _sc_scatter(e, dst2d, zeros_nd)
        nxt = (l + 1) % _L
        n, ps, pd = _tc_node(n, parts, w1n[l], w1a[l], nb1[l], node_w2[l],
                             nb2[l], nls[l], nlb[l], w1s[nxt], w1d[nxt])
    return (n, e)
